# Initial kernel scaffold; baseline (speedup 1.0000x reference)
import jax
import jax.numpy as jnp
from jax.experimental import pallas as pl


def kernel(x, edge_index, fc0_w, fc0_b, w1, u1, c1, b1, w2, u2, c2, b2, w3, u3, c3, b3, fc1_w, fc1_b, fc2_w, fc2_b):
    raise NotImplementedError("write your pallas kernel here")


# speedup vs baseline: 1.1450x; 1.1450x over previous
import functools
import jax
import jax.numpy as jnp
from jax import lax
from jax.experimental import pallas as pl
from jax.experimental.pallas import tpu as pltpu

_N = 10000
_E = 320000
_H = 8


def _elu(v):
    return jnp.where(v > 0, v, jnp.exp(jnp.minimum(v, 0.0)) - 1.0)


def _stage_in_body(x_ref, w_ref, b_ref, u_ref, c_ref, h_ref, a_ref, ac_ref):
    h = _elu(jnp.dot(x_ref[...], w_ref[...],
                     preferred_element_type=jnp.float32) + b_ref[...])
    h_ref[...] = h
    a = jnp.dot(h, u_ref[...], preferred_element_type=jnp.float32)
    a_ref[...] = a
    ac_ref[...] = a + c_ref[...]


def _stage_in(x, w, b, u, c):
    n = x.shape[0]
    return pl.pallas_call(
        _stage_in_body,
        out_shape=(
            jax.ShapeDtypeStruct((n, w.shape[1]), jnp.float32),
            jax.ShapeDtypeStruct((n, _H), jnp.float32),
            jax.ShapeDtypeStruct((n, _H), jnp.float32),
        ),
    )(x, w, b.reshape(1, -1), u, c.reshape(1, -1))


def _stage_mid_body(z_ref, hprev_ref, cnt_ref, wr_ref, b_ref, c_ref,
                    u_ref, cn_ref, h_ref, a_ref, ac_ref, *, heads_c):
    heads, cin = heads_c
    ql = jax.nn.softmax(c_ref[...], axis=-1)
    wr = wr_ref[...]
    out_c = wr.shape[1]
    weff = jnp.sum(wr.reshape(heads, cin, out_c) * ql.reshape(heads, 1, 1),
                   axis=0)
    agg = (jnp.dot(z_ref[...], wr, preferred_element_type=jnp.float32)
           + jnp.dot(hprev_ref[...], weff, preferred_element_type=jnp.float32))
    agg = agg / jnp.clip(cnt_ref[...], 1.0, None)
    h = _elu(agg + b_ref[...])
    h_ref[...] = h
    a = jnp.dot(h, u_ref[...], preferred_element_type=jnp.float32)
    a_ref[...] = a
    ac_ref[...] = a + cn_ref[...]


def _stage_mid(z, hprev, cnt, wr, b, c, u_next, c_next):
    n = z.shape[0]
    heads_c = (_H, hprev.shape[1])
    return pl.pallas_call(
        functools.partial(_stage_mid_body, heads_c=heads_c),
        out_shape=(
            jax.ShapeDtypeStruct((n, wr.shape[1]), jnp.float32),
            jax.ShapeDtypeStruct((n, _H), jnp.float32),
            jax.ShapeDtypeStruct((n, _H), jnp.float32),
        ),
    )(z, hprev, cnt.reshape(-1, 1), wr, b.reshape(1, -1), c.reshape(1, -1),
      u_next, c_next.reshape(1, -1))


def _stage_out_body(z_ref, hprev_ref, cnt_ref, wr_ref, b_ref, c_ref,
                    w1_ref, b1_ref, w2_ref, b2_ref, o_ref, *, heads_c):
    heads, cin = heads_c
    ql = jax.nn.softmax(c_ref[...], axis=-1)
    wr = wr_ref[...]
    out_c = wr.shape[1]
    weff = jnp.sum(wr.reshape(heads, cin, out_c) * ql.reshape(heads, 1, 1),
                   axis=0)
    agg = (jnp.dot(z_ref[...], wr, preferred_element_type=jnp.float32)
           + jnp.dot(hprev_ref[...], weff, preferred_element_type=jnp.float32))
    agg = agg / jnp.clip(cnt_ref[...], 1.0, None)
    h = _elu(agg + b_ref[...])
    h = _elu(jnp.dot(h, w1_ref[...], preferred_element_type=jnp.float32)
             + b1_ref[...])
    logits = (jnp.dot(h, w2_ref[...], preferred_element_type=jnp.float32)
              + b2_ref[...])
    m = jnp.max(logits, axis=-1, keepdims=True)
    s = jnp.log(jnp.sum(jnp.exp(logits - m), axis=-1, keepdims=True))
    o_ref[...] = logits - m - s


def _stage_out(z, hprev, cnt, wr, b, c, fc1_w, fc1_b, fc2_w, fc2_b):
    n = z.shape[0]
    heads_c = (_H, hprev.shape[1])
    return pl.pallas_call(
        functools.partial(_stage_out_body, heads_c=heads_c),
        out_shape=jax.ShapeDtypeStruct((n, fc2_w.shape[1]), jnp.float32),
    )(z, hprev, cnt.reshape(-1, 1), wr, b.reshape(1, -1), c.reshape(1, -1),
      fc1_w, fc1_b.reshape(1, -1), fc2_w, fc2_b.reshape(1, -1))


def _edge_agg(h, a, ac, src, dst, w_e):
    logits = ac[src] - a[dst]
    q = jax.nn.softmax(logits, axis=1) * w_e[:, None]
    c = h.shape[1]
    contrib = q[:, :, None] * h[src][:, None, :]
    z = jax.ops.segment_sum(contrib.reshape(-1, _H * c), dst, num_segments=_N)
    return z


def _rearrange_w(w, cin):
    out_c = w.shape[1] // _H
    return jnp.transpose(w.reshape(cin, _H, out_c), (1, 0, 2)).reshape(
        _H * cin, out_c)


def kernel(x, edge_index, fc0_w, fc0_b, w1, u1, c1, b1, w2, u2, c2, b2,
           w3, u3, c3, b3, fc1_w, fc1_b, fc2_w, fc2_b):
    src, dst = edge_index[0], edge_index[1]
    w_e = (src != dst).astype(jnp.float32)

    cnt = 1.0 + jax.ops.segment_sum(w_e, dst, num_segments=_N)

    wr1 = _rearrange_w(w1, 16)
    wr2 = _rearrange_w(w2, 32)
    wr3 = _rearrange_w(w3, 64)

    h0, a1, ac1 = _stage_in(x, fc0_w, fc0_b, u1, c1)
    z1 = _edge_agg(h0, a1, ac1, src, dst, w_e)
    h1, a2, ac2 = _stage_mid(z1, h0, cnt, wr1, b1, c1, u2, c2)
    z2 = _edge_agg(h1, a2, ac2, src, dst, w_e)
    h2, a3, ac3 = _stage_mid(z2, h1, cnt, wr2, b2, c2, u3, c3)
    z3 = _edge_agg(h2, a3, ac3, src, dst, w_e)
    return _stage_out(z3, h2, cnt, wr3, b3, c3, fc1_w, fc1_b, fc2_w, fc2_b)


# speedup vs baseline: 3.6666x; 3.2022x over previous
import functools
import jax
import jax.numpy as jnp
from jax import lax
from jax.experimental import pallas as pl
from jax.experimental.pallas import tpu as pltpu
from jax.experimental.pallas import tpu_sc as plsc

_N = 10000
_E = 320000
_H = 8

_NC, _NS = 2, 16
_NW = _NC * _NS
_NB = 64
_RNG = 157
_NPAD = _NB * _RNG
_KCH = 128
_EPW = 10112
_EPAD = _NW * _EPW
_NCHP = _EPW // _KCH
_CAP = 320512
_BE = _NB * _CAP
_MAGIC = 26716
_STG = 11264


def _wid():
    return lax.axis_index("c") * _NS + lax.axis_index("s")


def _sc_mesh():
    return plsc.VectorSubcoreMesh(core_axis_name="c", subcore_axis_name="s",
                                  num_cores=_NC, num_subcores=_NS)


def _iota16():
    return lax.iota(jnp.int32, 16)


def _pad16(v):
    return jnp.bitwise_and(v + 15, -16)


def _b1_body(src_hbm, dst_hbm, counts_hbm, sbuf, dbuf, cntbuf, cnt_s):
    w = _wid()

    def zero(b, _):
        cnt_s[b] = 0
        return 0
    lax.fori_loop(0, _NB, zero, 0)

    base = w * _EPW

    def chunk(j, _):
        pltpu.sync_copy(src_hbm.at[pl.ds(base + j * _KCH, _KCH)], sbuf)
        pltpu.sync_copy(dst_hbm.at[pl.ds(base + j * _KCH, _KCH)], dbuf)

        def grp(g, _):
            sv = sbuf[pl.ds(g * 16, 16)]
            dv = dbuf[pl.ds(g * 16, 16)]
            bv = lax.shift_right_logical(dv * _MAGIC, 22)
            tv = jnp.where(sv != dv, bv, _NB)
            for k in range(16):
                b = tv[k]
                cnt_s[b] = cnt_s[b] + 1
            return 0
        lax.fori_loop(0, _KCH // 16, grp, 0)
        return 0
    lax.fori_loop(0, _NCHP, chunk, 0)

    for i4 in range(_NB // 16):
        acc = jnp.zeros((16,), jnp.int32)
        for k in range(16):
            acc = jnp.where(_iota16() == k,
                            jnp.full((16,), cnt_s[i4 * 16 + k], jnp.int32),
                            acc)
        cntbuf[pl.ds(i4 * 16, 16)] = acc
    pltpu.sync_copy(cntbuf, counts_hbm.at[pl.ds(w * _NB, _NB)])


def _bin_count(src_pad, dst_pad):
    return pl.kernel(
        _b1_body,
        out_type=jax.ShapeDtypeStruct((_NW * _NB,), jnp.int32),
        mesh=_sc_mesh(),
        scratch_types=[
            pltpu.VMEM((_KCH,), jnp.int32),
            pltpu.VMEM((_KCH,), jnp.int32),
            pltpu.VMEM((_NB,), jnp.int32),
            pltpu.SMEM((_NB + 1,), jnp.int32),
        ],
    )(src_pad, dst_pad)


def _b2_body(src_hbm, dst_hbm, counts_hbm, bpk_hbm,
             sbuf, dbuf, cbuf, staging,
             gseg_s, lseg_s, fill_s, mini_s, semf):
    w = _wid()
    pltpu.sync_copy(counts_hbm, cbuf)

    for bg in range(_NB // 16):
        acc0 = (_iota16() + bg * 16) * _CAP

        def accp(p, a):
            return a + _pad16(cbuf[pl.ds(p * _NB + bg * 16, 16)])
        acc = lax.fori_loop(0, w, accp, acc0)
        myc = _pad16(cbuf[pl.ds(w * _NB + bg * 16, 16)])
        for k in range(16):
            gseg_s[bg * 16 + k] = acc[k]
            lseg_s[bg * 16 + k] = myc[k]

    def lpre(b, run):
        t = lseg_s[b]
        lseg_s[b] = run
        return run + t
    lax.fori_loop(0, _NB, lpre, 0)

    def zf(b, _):
        fill_s[b] = 0
        return 0
    lax.fori_loop(0, _NB + 1, zf, 0)

    base = w * _EPW

    def chunk(j, _):
        pltpu.sync_copy(src_hbm.at[pl.ds(base + j * _KCH, _KCH)], sbuf)
        pltpu.sync_copy(dst_hbm.at[pl.ds(base + j * _KCH, _KCH)], dbuf)

        def grp(g, _):
            sv = sbuf[pl.ds(g * 16, 16)]
            dv = dbuf[pl.ds(g * 16, 16)]
            bv = lax.shift_right_logical(dv * _MAGIC, 22)
            tv = jnp.where(sv != dv, bv, _NB)
            pkv = jnp.bitwise_or(lax.shift_left(sv, 14), dv)
            for k in range(16):
                b1 = tv[k]
                fc = fill_s[b1]
                mini_s[b1 * 16 + jnp.bitwise_and(fc, 15)] = pkv[k]
                fill_s[b1] = fc + 1
                do_flush = jnp.logical_and(jnp.bitwise_and(fc, 15) == 15,
                                           b1 < _NB)

                @pl.when(do_flush)
                def _():
                    acc = jnp.zeros((16,), jnp.int32)
                    for kk in range(16):
                        acc = jnp.where(
                            _iota16() == kk,
                            jnp.full((16,), mini_s[b1 * 16 + kk], jnp.int32),
                            acc)
                    spos = lseg_s[b1] + jnp.bitwise_and(fc, -16)
                    staging[pl.ds(spos, 16)] = acc
            return 0
        lax.fori_loop(0, _KCH // 16, grp, 0)
        return 0
    lax.fori_loop(0, _NCHP, chunk, 0)

    def tail(b, _):
        fc = fill_s[b]
        rem = jnp.bitwise_and(fc, 15)

        @pl.when(rem > 0)
        def _():
            sent = b * 157 * 16385
            acc = jnp.zeros((16,), jnp.int32)
            for kk in range(16):
                val = jnp.where(kk < rem, mini_s[b * 16 + kk], sent)
                acc = jnp.where(_iota16() == kk,
                                jnp.full((16,), val, jnp.int32), acc)
            spos = lseg_s[b] + jnp.bitwise_and(fc, -16)
            staging[pl.ds(spos, 16)] = acc
        return 0
    lax.fori_loop(0, _NB, tail, 0)

    def fire(b, _):
        nb16 = lax.shift_right_logical(fill_s[b] + 15, 4)
        ls = pl.multiple_of(lseg_s[b], 16)
        gs = pl.multiple_of(gseg_s[b], 16)

        def one(m, _):
            pltpu.async_copy(
                staging.at[pl.ds(ls + m * 16, 16)],
                bpk_hbm.at[pl.ds(gs + m * 16, 16)], semf)
            return 0
        lax.fori_loop(0, nb16, one, 0)
        return 0
    lax.fori_loop(0, _NB, fire, 0)

    def drain(b, _):
        nb16 = lax.shift_right_logical(fill_s[b] + 15, 4)
        ls = pl.multiple_of(lseg_s[b], 16)
        gs = pl.multiple_of(gseg_s[b], 16)

        def one(m, _):
            pltpu.make_async_copy(
                staging.at[pl.ds(ls + m * 16, 16)],
                bpk_hbm.at[pl.ds(gs + m * 16, 16)], semf).wait()
            return 0
        lax.fori_loop(0, nb16, one, 0)
        return 0
    lax.fori_loop(0, _NB, drain, 0)


def _bin_place(src_pad, dst_pad, counts):
    return pl.kernel(
        _b2_body,
        out_type=jax.ShapeDtypeStruct((_BE,), jnp.int32),
        mesh=_sc_mesh(),
        scratch_types=[
            pltpu.VMEM((_KCH,), jnp.int32),
            pltpu.VMEM((_KCH,), jnp.int32),
            pltpu.VMEM((_NW * _NB,), jnp.int32),
            pltpu.VMEM((_STG,), jnp.int32),
            pltpu.SMEM((_NB,), jnp.int32),
            pltpu.SMEM((_NB,), jnp.int32),
            pltpu.SMEM((_NB + 1,), jnp.int32),
            pltpu.SMEM(((_NB + 1) * 16,), jnp.int32),
            pltpu.SemaphoreType.DMA,
        ],
    )(src_pad, dst_pad, counts)


def _conv_body(bpk, counts_hbm, t_hbm, a_hbm, zrows0, drows0,
               z_hbm, deg_hbm, pbuf, idxs, dlbuf, webuf, tr, adst,
               zloc, degloc, cbuf, tots_s, sem, *, cdim, with_deg):
    w = _wid()
    hc = _H * cdim
    pltpu.sync_copy(counts_hbm, cbuf)

    bg = (2 * w) // 16

    def accp(p, a):
        return a + _pad16(cbuf[pl.ds(p * _NB + bg * 16, 16)])
    tots16 = lax.fori_loop(0, _NW, accp, jnp.zeros((16,), jnp.int32))
    for k in range(16):
        tots_s[k] = tots16[k]

    onehot0 = jnp.where(_iota16() == 0, 1.0, 0.0).astype(jnp.float32)
    head_m = _iota16() < _H

    def bucket(bk, _):
        b = 2 * w + bk
        ptot = tots_s[b - bg * 16]

        pltpu.sync_copy(zrows0, zloc)
        if with_deg:
            pltpu.sync_copy(drows0, degloc)
        pltpu.sync_copy(a_hbm.at[pl.ds(b * _RNG * 16, _RNG * 16)], adst)

        nch = lax.shift_right_logical(ptot + _KCH - 1, 7)

        def chunk(j, _):
            st = b * _CAP + j * _KCH
            pltpu.sync_copy(bpk.at[pl.ds(st, _KCH)], pbuf)

            def sgrp(g, _):
                pos = _iota16() + (g * 16 + j * _KCH)
                mval = pos < ptot
                pk = pbuf[pl.ds(g * 16, 16)]
                sv = lax.shift_right_logical(pk, 14)
                dv = jnp.bitwise_and(pk, 16383)
                wev = jnp.logical_and(mval, sv != dv)
                webuf[pl.ds(g * 16, 16)] = jnp.where(wev, 1.0, 0.0)
                idxs[pl.ds(g * 16, 16)] = jnp.where(mval, sv, 0)
                dlbuf[pl.ds(g * 16, 16)] = jnp.clip(dv - b * _RNG, 0,
                                                    _RNG - 1)
                return 0
            lax.fori_loop(0, _KCH // 16, sgrp, 0)

            pltpu.async_copy(t_hbm.at[idxs], tr, sem).wait()

            def egrp(g, _):
                dv16 = dlbuf[pl.ds(g * 16, 16)]
                wev16 = webuf[pl.ds(g * 16, 16)]
                for k in range(16):
                    e = g * 16 + k
                    dl = dv16[k]
                    lv = (tr[e, pl.ds(cdim, 16)]
                          - adst[pl.ds(dl * 16, 16)])
                    m0 = lv[0]
                    for hh in range(1, _H):
                        m0 = jnp.maximum(m0, lv[hh])
                    ev = jnp.where(head_m,
                                   jnp.exp(lv - jnp.full((16,), m0,
                                                         jnp.float32)),
                                   0.0)
                    s0 = ev[0]
                    for hh in range(1, _H):
                        s0 = s0 + ev[hh]
                    qfac = (jnp.full((16,), wev16[k], jnp.float32)
                            / jnp.full((16,), s0, jnp.float32))
                    hvs = [tr[e, pl.ds(16 * k2, 16)]
                           for k2 in range(cdim // 16)]
                    zb = dl * hc
                    for hh in range(_H):
                        qv = jnp.full((16,), ev[hh], jnp.float32) * qfac
                        for k2 in range(cdim // 16):
                            plsc.addupdate(
                                zloc.at[pl.ds(zb + hh * cdim + 16 * k2, 16)],
                                qv * hvs[k2])
                    if with_deg:
                        plsc.addupdate(degloc.at[pl.ds(dl * 16, 16)],
                                       onehot0 * wev16[k])
                return 0
            lax.fori_loop(0, _KCH // 16, egrp, 0)
            return 0
        lax.fori_loop(0, nch, chunk, 0)

        pltpu.sync_copy(zloc, z_hbm.at[pl.ds(b * _RNG * hc, _RNG * hc)])
        if with_deg:
            pltpu.sync_copy(degloc,
                            deg_hbm.at[pl.ds(b * _RNG * 16, _RNG * 16)])
        return 0
    lax.fori_loop(0, 2, bucket, 0)


def _edge_agg_sc(bpk, counts, t_tab, a_tab, cdim, with_deg):
    hc = _H * cdim
    zrows0 = jnp.zeros((_RNG * hc,), jnp.float32)
    drows0 = jnp.zeros((_RNG * 16,), jnp.float32)
    out_type = (jax.ShapeDtypeStruct((_NPAD * hc,), jnp.float32),
                jax.ShapeDtypeStruct((_NPAD * 16,), jnp.float32))
    scratch = [
        pltpu.VMEM((_KCH,), jnp.int32),
        pltpu.VMEM((_KCH,), jnp.int32),
        pltpu.VMEM((_KCH,), jnp.int32),
        pltpu.VMEM((_KCH,), jnp.float32),
        pltpu.VMEM((_KCH, 128), jnp.float32),
        pltpu.VMEM((_RNG * 16,), jnp.float32),
        pltpu.VMEM((_RNG * hc,), jnp.float32),
        pltpu.VMEM((_RNG * 16,), jnp.float32),
        pltpu.VMEM((_NW * _NB,), jnp.int32),
        pltpu.SMEM((16,), jnp.int32),
        pltpu.SemaphoreType.DMA,
    ]
    body = functools.partial(_conv_body, cdim=cdim, with_deg=with_deg)
    z, deg = pl.kernel(
        body, out_type=out_type, mesh=_sc_mesh(),
        scratch_types=scratch,
    )(bpk, counts, t_tab, a_tab, zrows0, drows0)
    return z.reshape(_NPAD, hc), deg.reshape(_NPAD, 16)


def _elu(v):
    return jnp.where(v > 0, v, jnp.exp(jnp.minimum(v, 0.0)) - 1.0)


def _ttab(h, a_plus_c, cdim):
    n = h.shape[0]
    pad = jnp.zeros((n, 128 - cdim - _H), jnp.float32)
    return jnp.concatenate([h, a_plus_c, pad], axis=1)


def _stage_in_body(x_ref, w_ref, b_ref, u_ref, c_ref, h_ref, t_ref, a_ref,
                   *, cdim):
    h = _elu(jnp.dot(x_ref[...], w_ref[...],
                     preferred_element_type=jnp.float32) + b_ref[...])
    h_ref[...] = h
    a = jnp.dot(h, u_ref[...], preferred_element_type=jnp.float32)
    t_ref[...] = _ttab(h, a + c_ref[...], cdim)
    a_ref[...] = jnp.concatenate(
        [a, jnp.zeros((h.shape[0], 16 - _H), jnp.float32)], axis=1)


def _stage_in(x, w, b, u, c):
    n = x.shape[0]
    cdim = w.shape[1]
    return pl.pallas_call(
        functools.partial(_stage_in_body, cdim=cdim),
        out_shape=(
            jax.ShapeDtypeStruct((n, cdim), jnp.float32),
            jax.ShapeDtypeStruct((n, 128), jnp.float32),
            jax.ShapeDtypeStruct((n, 16), jnp.float32),
        ),
    )(x, w, b.reshape(1, -1), u, c.reshape(1, -1))


def _stage_mid_body(z_ref, hprev_ref, deg_ref, wr_ref, b_ref, c_ref,
                    u_ref, cn_ref, h_ref, t_ref, a_ref, *, heads_c):
    heads, cin, cout = heads_c
    ql = jax.nn.softmax(c_ref[...][:, :heads], axis=-1)
    wr = wr_ref[...]
    weff = jnp.sum(wr.reshape(heads, cin, cout) * ql.reshape(heads, 1, 1),
                   axis=0)
    agg = (jnp.dot(z_ref[...], wr, preferred_element_type=jnp.float32)
           + jnp.dot(hprev_ref[...], weff, preferred_element_type=jnp.float32))
    cnt = 1.0 + deg_ref[...][:, 0:1]
    agg = agg / jnp.clip(cnt, 1.0, None)
    h = _elu(agg + b_ref[...])
    h_ref[...] = h
    a = jnp.dot(h, u_ref[...], preferred_element_type=jnp.float32)
    t_ref[...] = _ttab(h, a + cn_ref[...], cout)
    a_ref[...] = jnp.concatenate(
        [a, jnp.zeros((h.shape[0], 16 - _H), jnp.float32)], axis=1)


def _stage_mid(z, hprev, deg, wr, b, c, u_next, c_next):
    n = z.shape[0]
    heads_c = (_H, hprev.shape[1], wr.shape[1])
    return pl.pallas_call(
        functools.partial(_stage_mid_body, heads_c=heads_c),
        out_shape=(
            jax.ShapeDtypeStruct((n, wr.shape[1]), jnp.float32),
            jax.ShapeDtypeStruct((n, 128), jnp.float32),
            jax.ShapeDtypeStruct((n, 16), jnp.float32),
        ),
    )(z, hprev, deg, wr, b.reshape(1, -1), c.reshape(1, -1), u_next,
      c_next.reshape(1, -1))


def _stage_out_body(z_ref, hprev_ref, deg_ref, wr_ref, b_ref, c_ref,
                    w1_ref, b1_ref, w2_ref, b2_ref, o_ref, *, heads_c):
    heads, cin, cout = heads_c
    ql = jax.nn.softmax(c_ref[...][:, :heads], axis=-1)
    wr = wr_ref[...]
    weff = jnp.sum(wr.reshape(heads, cin, cout) * ql.reshape(heads, 1, 1),
                   axis=0)
    agg = (jnp.dot(z_ref[...], wr, preferred_element_type=jnp.float32)
           + jnp.dot(hprev_ref[...], weff, preferred_element_type=jnp.float32))
    cnt = 1.0 + deg_ref[...][:, 0:1]
    agg = agg / jnp.clip(cnt, 1.0, None)
    h = _elu(agg + b_ref[...])
    h = _elu(jnp.dot(h, w1_ref[...], preferred_element_type=jnp.float32)
             + b1_ref[...])
    logits = (jnp.dot(h, w2_ref[...], preferred_element_type=jnp.float32)
              + b2_ref[...])
    m = jnp.max(logits, axis=-1, keepdims=True)
    s = jnp.log(jnp.sum(jnp.exp(logits - m), axis=-1, keepdims=True))
    o_ref[...] = logits - m - s


def _stage_out(z, hprev, deg, wr, b, c, fc1_w, fc1_b, fc2_w, fc2_b):
    n = z.shape[0]
    heads_c = (_H, hprev.shape[1], wr.shape[1])
    return pl.pallas_call(
        functools.partial(_stage_out_body, heads_c=heads_c),
        out_shape=jax.ShapeDtypeStruct((n, fc2_w.shape[1]), jnp.float32),
    )(z, hprev, deg, wr, b.reshape(1, -1), c.reshape(1, -1), fc1_w,
      fc1_b.reshape(1, -1), fc2_w, fc2_b.reshape(1, -1))


def _rearrange_w(w, cin):
    out_c = w.shape[1] // _H
    return jnp.transpose(w.reshape(cin, _H, out_c), (1, 0, 2)).reshape(
        _H * cin, out_c)


def kernel(x, edge_index, fc0_w, fc0_b, w1, u1, c1, b1, w2, u2, c2, b2,
           w3, u3, c3, b3, fc1_w, fc1_b, fc2_w, fc2_b):
    src = jnp.pad(edge_index[0], (0, _EPAD - _E))
    dst = jnp.pad(edge_index[1], (0, _EPAD - _E))
    x_pad = jnp.pad(x, ((0, _NPAD - _N), (0, 0)))

    counts = _bin_count(src, dst)
    bpk = _bin_place(src, dst, counts)

    wr1 = _rearrange_w(w1, 16)
    wr2 = _rearrange_w(w2, 32)
    wr3 = _rearrange_w(w3, 64)

    h0, t1, a1 = _stage_in(x_pad, fc0_w, fc0_b, u1, c1)
    z1, deg = _edge_agg_sc(bpk, counts, t1, a1.reshape(-1), 16, True)
    h1, t2, a2 = _stage_mid(z1, h0, deg, wr1, b1, c1, u2, c2)
    z2, _ = _edge_agg_sc(bpk, counts, t2, a2.reshape(-1), 32, False)
    h2, t3, a3 = _stage_mid(z2, h1, deg, wr2, b2, c2, u3, c3)
    z3, _ = _edge_agg_sc(bpk, counts, t3, a3.reshape(-1), 64, False)
    out = _stage_out(z3, h2, deg, wr3, b3, c3, fc1_w, fc1_b, fc2_w, fc2_b)
    return out[:_N]


# speedup vs baseline: 4.3571x; 1.1883x over previous
import functools
import jax
import jax.numpy as jnp
from jax import lax
from jax.experimental import pallas as pl
from jax.experimental.pallas import tpu as pltpu
from jax.experimental.pallas import tpu_sc as plsc

_N = 10000
_E = 320000
_H = 8

_NC, _NS = 2, 16
_NW = _NC * _NS
_NB = 64
_RNG = 157
_NPAD = _NB * _RNG
_KCH = 128
_EPW = 10112
_EPAD = _NW * _EPW
_NCHP = _EPW // _KCH
_CAP = 320512
_BE = _NB * _CAP
_MAGIC = 26716
_STG = 11264


def _wid():
    return lax.axis_index("c") * _NS + lax.axis_index("s")


def _sc_mesh():
    return plsc.VectorSubcoreMesh(core_axis_name="c", subcore_axis_name="s",
                                  num_cores=_NC, num_subcores=_NS)


def _iota16():
    return lax.iota(jnp.int32, 16)


def _lgather(v, idx):
    return v.at[idx].get(mode='promise_in_bounds')


def _bmax8(v):
    for r in (4, 2, 1):
        idx = jnp.bitwise_and(_iota16() + r, 7)
        v = jnp.maximum(v, _lgather(v, idx))
    return v


def _bsum8(v):
    for r in (4, 2, 1):
        idx = jnp.bitwise_and(_iota16() + r, 7)
        v = v + _lgather(v, idx)
    return v


def _bcast(v, k):
    return _lgather(v, jnp.full((16,), k, jnp.int32))


def _pad16(v):
    return jnp.bitwise_and(v + 15, -16)


def _b1_body(src_hbm, dst_hbm, counts_hbm, sbuf, dbuf, cntbuf, cnt_s):
    w = _wid()

    def zero(b, _):
        cnt_s[b] = 0
        return 0
    lax.fori_loop(0, _NB, zero, 0)

    base = w * _EPW

    def chunk(j, _):
        pltpu.sync_copy(src_hbm.at[pl.ds(base + j * _KCH, _KCH)], sbuf)
        pltpu.sync_copy(dst_hbm.at[pl.ds(base + j * _KCH, _KCH)], dbuf)

        def grp(g, _):
            sv = sbuf[pl.ds(g * 16, 16)]
            dv = dbuf[pl.ds(g * 16, 16)]
            bv = lax.shift_right_logical(dv * _MAGIC, 22)
            tv = jnp.where(sv != dv, bv, _NB)
            for k in range(16):
                b = tv[k]
                cnt_s[b] = cnt_s[b] + 1
            return 0
        lax.fori_loop(0, _KCH // 16, grp, 0)
        return 0
    lax.fori_loop(0, _NCHP, chunk, 0)

    for i4 in range(_NB // 16):
        acc = jnp.zeros((16,), jnp.int32)
        for k in range(16):
            acc = jnp.where(_iota16() == k,
                            jnp.full((16,), cnt_s[i4 * 16 + k], jnp.int32),
                            acc)
        cntbuf[pl.ds(i4 * 16, 16)] = acc
    pltpu.sync_copy(cntbuf, counts_hbm.at[pl.ds(w * _NB, _NB)])


def _bin_count(src_pad, dst_pad):
    return pl.kernel(
        _b1_body,
        out_type=jax.ShapeDtypeStruct((_NW * _NB,), jnp.int32),
        mesh=_sc_mesh(),
        scratch_types=[
            pltpu.VMEM((_KCH,), jnp.int32),
            pltpu.VMEM((_KCH,), jnp.int32),
            pltpu.VMEM((_NB,), jnp.int32),
            pltpu.SMEM((_NB + 1,), jnp.int32),
        ],
    )(src_pad, dst_pad)


def _b2_body(src_hbm, dst_hbm, counts_hbm, bpk_hbm,
             sbuf, dbuf, cbuf, staging,
             gseg_s, lseg_s, fill_s, mini_s, semf):
    w = _wid()
    pltpu.sync_copy(counts_hbm, cbuf)

    for bg in range(_NB // 16):
        acc0 = (_iota16() + bg * 16) * _CAP

        def accp(p, a):
            return a + _pad16(cbuf[pl.ds(p * _NB + bg * 16, 16)])
        acc = lax.fori_loop(0, w, accp, acc0)
        myc = _pad16(cbuf[pl.ds(w * _NB + bg * 16, 16)])
        for k in range(16):
            gseg_s[bg * 16 + k] = acc[k]
            lseg_s[bg * 16 + k] = myc[k]

    def lpre(b, run):
        t = lseg_s[b]
        lseg_s[b] = run
        return run + t
    lax.fori_loop(0, _NB, lpre, 0)

    def zf(b, _):
        fill_s[b] = 0
        return 0
    lax.fori_loop(0, _NB + 1, zf, 0)

    base = w * _EPW

    def chunk(j, _):
        pltpu.sync_copy(src_hbm.at[pl.ds(base + j * _KCH, _KCH)], sbuf)
        pltpu.sync_copy(dst_hbm.at[pl.ds(base + j * _KCH, _KCH)], dbuf)

        def grp(g, _):
            sv = sbuf[pl.ds(g * 16, 16)]
            dv = dbuf[pl.ds(g * 16, 16)]
            bv = lax.shift_right_logical(dv * _MAGIC, 22)
            tv = jnp.where(sv != dv, bv, _NB)
            pkv = jnp.bitwise_or(lax.shift_left(sv, 14), dv)
            for k in range(16):
                b1 = tv[k]
                fc = fill_s[b1]
                mini_s[b1 * 16 + jnp.bitwise_and(fc, 15)] = pkv[k]
                fill_s[b1] = fc + 1
                do_flush = jnp.logical_and(jnp.bitwise_and(fc, 15) == 15,
                                           b1 < _NB)

                @pl.when(do_flush)
                def _():
                    acc = jnp.zeros((16,), jnp.int32)
                    for kk in range(16):
                        acc = jnp.where(
                            _iota16() == kk,
                            jnp.full((16,), mini_s[b1 * 16 + kk], jnp.int32),
                            acc)
                    spos = lseg_s[b1] + jnp.bitwise_and(fc, -16)
                    staging[pl.ds(spos, 16)] = acc
            return 0
        lax.fori_loop(0, _KCH // 16, grp, 0)
        return 0
    lax.fori_loop(0, _NCHP, chunk, 0)

    def tail(b, _):
        fc = fill_s[b]
        rem = jnp.bitwise_and(fc, 15)

        @pl.when(rem > 0)
        def _():
            sent = b * 157 * 16385
            acc = jnp.zeros((16,), jnp.int32)
            for kk in range(16):
                val = jnp.where(kk < rem, mini_s[b * 16 + kk], sent)
                acc = jnp.where(_iota16() == kk,
                                jnp.full((16,), val, jnp.int32), acc)
            spos = lseg_s[b] + jnp.bitwise_and(fc, -16)
            staging[pl.ds(spos, 16)] = acc
        return 0
    lax.fori_loop(0, _NB, tail, 0)

    def fire(b, _):
        nb16 = lax.shift_right_logical(fill_s[b] + 15, 4)
        ls = pl.multiple_of(lseg_s[b], 16)
        gs = pl.multiple_of(gseg_s[b], 16)

        def one(m, _):
            pltpu.async_copy(
                staging.at[pl.ds(ls + m * 16, 16)],
                bpk_hbm.at[pl.ds(gs + m * 16, 16)], semf)
            return 0
        lax.fori_loop(0, nb16, one, 0)
        return 0
    lax.fori_loop(0, _NB, fire, 0)

    def drain(b, _):
        nb16 = lax.shift_right_logical(fill_s[b] + 15, 4)
        ls = pl.multiple_of(lseg_s[b], 16)
        gs = pl.multiple_of(gseg_s[b], 16)

        def one(m, _):
            pltpu.make_async_copy(
                staging.at[pl.ds(ls + m * 16, 16)],
                bpk_hbm.at[pl.ds(gs + m * 16, 16)], semf).wait()
            return 0
        lax.fori_loop(0, nb16, one, 0)
        return 0
    lax.fori_loop(0, _NB, drain, 0)


def _bin_place(src_pad, dst_pad, counts):
    return pl.kernel(
        _b2_body,
        out_type=jax.ShapeDtypeStruct((_BE,), jnp.int32),
        mesh=_sc_mesh(),
        scratch_types=[
            pltpu.VMEM((_KCH,), jnp.int32),
            pltpu.VMEM((_KCH,), jnp.int32),
            pltpu.VMEM((_NW * _NB,), jnp.int32),
            pltpu.VMEM((_STG,), jnp.int32),
            pltpu.SMEM((_NB,), jnp.int32),
            pltpu.SMEM((_NB,), jnp.int32),
            pltpu.SMEM((_NB + 1,), jnp.int32),
            pltpu.SMEM(((_NB + 1) * 16,), jnp.int32),
            pltpu.SemaphoreType.DMA,
        ],
    )(src_pad, dst_pad, counts)


def _conv_body(bpk, counts_hbm, t_hbm, a_hbm, zrows0, drows0,
               z_hbm, deg_hbm, pbuf, idxs, dlbuf, webuf, tr, adst,
               zloc, degloc, cbuf, tots_s, sem, *, cdim, with_deg):
    w = _wid()
    hc = _H * cdim
    pltpu.sync_copy(counts_hbm, cbuf)

    bg = (2 * w) // 16

    def accp(p, a):
        return a + _pad16(cbuf[pl.ds(p * _NB + bg * 16, 16)])
    tots16 = lax.fori_loop(0, _NW, accp, jnp.zeros((16,), jnp.int32))
    for k in range(16):
        tots_s[k] = tots16[k]

    onehot0 = jnp.where(_iota16() == 0, 1.0, 0.0).astype(jnp.float32)
    head_m = _iota16() < _H

    def bucket(bk, _):
        b = 2 * w + bk
        ptot = tots_s[b - bg * 16]

        pltpu.sync_copy(zrows0, zloc)
        if with_deg:
            pltpu.sync_copy(drows0, degloc)
        pltpu.sync_copy(a_hbm.at[pl.ds(b * _RNG * 16, _RNG * 16)], adst)

        nch = lax.shift_right_logical(ptot + _KCH - 1, 7)

        def chunk(j, _):
            st = b * _CAP + j * _KCH
            pltpu.sync_copy(bpk.at[pl.ds(st, _KCH)], pbuf)

            def sgrp(g, _):
                pos = _iota16() + (g * 16 + j * _KCH)
                mval = pos < ptot
                pk = pbuf[pl.ds(g * 16, 16)]
                sv = lax.shift_right_logical(pk, 14)
                dv = jnp.bitwise_and(pk, 16383)
                wev = jnp.logical_and(mval, sv != dv)
                webuf[pl.ds(g * 16, 16)] = jnp.where(wev, 1.0, 0.0)
                idxs[pl.ds(g * 16, 16)] = jnp.where(mval, sv, 0)
                dlbuf[pl.ds(g * 16, 16)] = jnp.clip(dv - b * _RNG, 0,
                                                    _RNG - 1)
                return 0
            lax.fori_loop(0, _KCH // 16, sgrp, 0)

            pltpu.async_copy(t_hbm.at[idxs], tr, sem).wait()

            def egrp(g, _):
                dv16 = dlbuf[pl.ds(g * 16, 16)]
                wev16 = webuf[pl.ds(g * 16, 16)]
                for k in range(16):
                    e = g * 16 + k
                    dl = dv16[k]
                    lv = (tr[e, pl.ds(cdim, 16)]
                          - adst[pl.ds(dl * 16, 16)])
                    mx = _bmax8(jnp.where(head_m, lv, -1e30))
                    ev = jnp.where(head_m, jnp.exp(lv - mx), 0.0)
                    s = _bsum8(ev)
                    wk = _bcast(wev16, k)
                    qall = ev * (wk / s)
                    hvs = [tr[e, pl.ds(16 * k2, 16)]
                           for k2 in range(cdim // 16)]
                    zb = dl * hc
                    for hh in range(_H):
                        qv = _bcast(qall, hh)
                        for k2 in range(cdim // 16):
                            plsc.addupdate(
                                zloc.at[pl.ds(zb + hh * cdim + 16 * k2, 16)],
                                qv * hvs[k2])
                    if with_deg:
                        plsc.addupdate(degloc.at[pl.ds(dl * 16, 16)],
                                       onehot0 * wk)
                return 0
            lax.fori_loop(0, _KCH // 16, egrp, 0)
            return 0
        lax.fori_loop(0, nch, chunk, 0)

        pltpu.sync_copy(zloc, z_hbm.at[pl.ds(b * _RNG * hc, _RNG * hc)])
        if with_deg:
            pltpu.sync_copy(degloc,
                            deg_hbm.at[pl.ds(b * _RNG * 16, _RNG * 16)])
        return 0
    lax.fori_loop(0, 2, bucket, 0)


def _edge_agg_sc(bpk, counts, t_tab, a_tab, cdim, with_deg):
    hc = _H * cdim
    zrows0 = jnp.zeros((_RNG * hc,), jnp.float32)
    drows0 = jnp.zeros((_RNG * 16,), jnp.float32)
    out_type = (jax.ShapeDtypeStruct((_NPAD * hc,), jnp.float32),
                jax.ShapeDtypeStruct((_NPAD * 16,), jnp.float32))
    scratch = [
        pltpu.VMEM((_KCH,), jnp.int32),
        pltpu.VMEM((_KCH,), jnp.int32),
        pltpu.VMEM((_KCH,), jnp.int32),
        pltpu.VMEM((_KCH,), jnp.float32),
        pltpu.VMEM((_KCH, 128), jnp.float32),
        pltpu.VMEM((_RNG * 16,), jnp.float32),
        pltpu.VMEM((_RNG * hc,), jnp.float32),
        pltpu.VMEM((_RNG * 16,), jnp.float32),
        pltpu.VMEM((_NW * _NB,), jnp.int32),
        pltpu.SMEM((16,), jnp.int32),
        pltpu.SemaphoreType.DMA,
    ]
    body = functools.partial(_conv_body, cdim=cdim, with_deg=with_deg)
    z, deg = pl.kernel(
        body, out_type=out_type, mesh=_sc_mesh(),
        scratch_types=scratch,
    )(bpk, counts, t_tab, a_tab, zrows0, drows0)
    return z.reshape(_NPAD, hc), deg.reshape(_NPAD, 16)


def _elu(v):
    return jnp.where(v > 0, v, jnp.exp(jnp.minimum(v, 0.0)) - 1.0)


def _ttab(h, a_plus_c, cdim):
    n = h.shape[0]
    pad = jnp.zeros((n, 128 - cdim - _H), jnp.float32)
    return jnp.concatenate([h, a_plus_c, pad], axis=1)


def _stage_in_body(x_ref, w_ref, b_ref, u_ref, c_ref, h_ref, t_ref, a_ref,
                   *, cdim):
    h = _elu(jnp.dot(x_ref[...], w_ref[...],
                     preferred_element_type=jnp.float32) + b_ref[...])
    h_ref[...] = h
    a = jnp.dot(h, u_ref[...], preferred_element_type=jnp.float32)
    t_ref[...] = _ttab(h, a + c_ref[...], cdim)
    a_ref[...] = jnp.concatenate(
        [a, jnp.zeros((h.shape[0], 16 - _H), jnp.float32)], axis=1)


def _stage_in(x, w, b, u, c):
    n = x.shape[0]
    cdim = w.shape[1]
    return pl.pallas_call(
        functools.partial(_stage_in_body, cdim=cdim),
        out_shape=(
            jax.ShapeDtypeStruct((n, cdim), jnp.float32),
            jax.ShapeDtypeStruct((n, 128), jnp.float32),
            jax.ShapeDtypeStruct((n, 16), jnp.float32),
        ),
    )(x, w, b.reshape(1, -1), u, c.reshape(1, -1))


def _stage_mid_body(z_ref, hprev_ref, deg_ref, wr_ref, b_ref, c_ref,
                    u_ref, cn_ref, h_ref, t_ref, a_ref, *, heads_c):
    heads, cin, cout = heads_c
    ql = jax.nn.softmax(c_ref[...][:, :heads], axis=-1)
    wr = wr_ref[...]
    weff = jnp.sum(wr.reshape(heads, cin, cout) * ql.reshape(heads, 1, 1),
                   axis=0)
    agg = (jnp.dot(z_ref[...], wr, preferred_element_type=jnp.float32)
           + jnp.dot(hprev_ref[...], weff, preferred_element_type=jnp.float32))
    cnt = 1.0 + deg_ref[...][:, 0:1]
    agg = agg / jnp.clip(cnt, 1.0, None)
    h = _elu(agg + b_ref[...])
    h_ref[...] = h
    a = jnp.dot(h, u_ref[...], preferred_element_type=jnp.float32)
    t_ref[...] = _ttab(h, a + cn_ref[...], cout)
    a_ref[...] = jnp.concatenate(
        [a, jnp.zeros((h.shape[0], 16 - _H), jnp.float32)], axis=1)


def _stage_mid(z, hprev, deg, wr, b, c, u_next, c_next):
    n = z.shape[0]
    heads_c = (_H, hprev.shape[1], wr.shape[1])
    return pl.pallas_call(
        functools.partial(_stage_mid_body, heads_c=heads_c),
        out_shape=(
            jax.ShapeDtypeStruct((n, wr.shape[1]), jnp.float32),
            jax.ShapeDtypeStruct((n, 128), jnp.float32),
            jax.ShapeDtypeStruct((n, 16), jnp.float32),
        ),
    )(z, hprev, deg, wr, b.reshape(1, -1), c.reshape(1, -1), u_next,
      c_next.reshape(1, -1))


def _stage_out_body(z_ref, hprev_ref, deg_ref, wr_ref, b_ref, c_ref,
                    w1_ref, b1_ref, w2_ref, b2_ref, o_ref, *, heads_c):
    heads, cin, cout = heads_c
    ql = jax.nn.softmax(c_ref[...][:, :heads], axis=-1)
    wr = wr_ref[...]
    weff = jnp.sum(wr.reshape(heads, cin, cout) * ql.reshape(heads, 1, 1),
                   axis=0)
    agg = (jnp.dot(z_ref[...], wr, preferred_element_type=jnp.float32)
           + jnp.dot(hprev_ref[...], weff, preferred_element_type=jnp.float32))
    cnt = 1.0 + deg_ref[...][:, 0:1]
    agg = agg / jnp.clip(cnt, 1.0, None)
    h = _elu(agg + b_ref[...])
    h = _elu(jnp.dot(h, w1_ref[...], preferred_element_type=jnp.float32)
             + b1_ref[...])
    logits = (jnp.dot(h, w2_ref[...], preferred_element_type=jnp.float32)
              + b2_ref[...])
    m = jnp.max(logits, axis=-1, keepdims=True)
    s = jnp.log(jnp.sum(jnp.exp(logits - m), axis=-1, keepdims=True))
    o_ref[...] = logits - m - s


def _stage_out(z, hprev, deg, wr, b, c, fc1_w, fc1_b, fc2_w, fc2_b):
    n = z.shape[0]
    heads_c = (_H, hprev.shape[1], wr.shape[1])
    return pl.pallas_call(
        functools.partial(_stage_out_body, heads_c=heads_c),
        out_shape=jax.ShapeDtypeStruct((n, fc2_w.shape[1]), jnp.float32),
    )(z, hprev, deg, wr, b.reshape(1, -1), c.reshape(1, -1), fc1_w,
      fc1_b.reshape(1, -1), fc2_w, fc2_b.reshape(1, -1))


def _rearrange_w(w, cin):
    out_c = w.shape[1] // _H
    return jnp.transpose(w.reshape(cin, _H, out_c), (1, 0, 2)).reshape(
        _H * cin, out_c)


def kernel(x, edge_index, fc0_w, fc0_b, w1, u1, c1, b1, w2, u2, c2, b2,
           w3, u3, c3, b3, fc1_w, fc1_b, fc2_w, fc2_b):
    src = jnp.pad(edge_index[0], (0, _EPAD - _E))
    dst = jnp.pad(edge_index[1], (0, _EPAD - _E))
    x_pad = jnp.pad(x, ((0, _NPAD - _N), (0, 0)))

    counts = _bin_count(src, dst)
    bpk = _bin_place(src, dst, counts)

    wr1 = _rearrange_w(w1, 16)
    wr2 = _rearrange_w(w2, 32)
    wr3 = _rearrange_w(w3, 64)

    h0, t1, a1 = _stage_in(x_pad, fc0_w, fc0_b, u1, c1)
    z1, deg = _edge_agg_sc(bpk, counts, t1, a1.reshape(-1), 16, True)
    h1, t2, a2 = _stage_mid(z1, h0, deg, wr1, b1, c1, u2, c2)
    z2, _ = _edge_agg_sc(bpk, counts, t2, a2.reshape(-1), 32, False)
    h2, t3, a3 = _stage_mid(z2, h1, deg, wr2, b2, c2, u3, c3)
    z3, _ = _edge_agg_sc(bpk, counts, t3, a3.reshape(-1), 64, False)
    out = _stage_out(z3, h2, deg, wr3, b3, c3, fc1_w, fc1_b, fc2_w, fc2_b)
    return out[:_N]


# speedup vs baseline: 4.3720x; 1.0034x over previous
import functools
import jax
import jax.numpy as jnp
from jax import lax
from jax.experimental import pallas as pl
from jax.experimental.pallas import tpu as pltpu
from jax.experimental.pallas import tpu_sc as plsc

_N = 10000
_E = 320000
_H = 8

_NC, _NS = 2, 16
_NW = _NC * _NS
_NB = 64
_RNG = 157
_NPAD = _NB * _RNG
_KCH = 128
_EPW = 10112
_EPAD = _NW * _EPW
_NCHP = _EPW // _KCH
_CAP = 320512
_BE = _NB * _CAP
_MAGIC = 26716
_STG = 11264


def _wid():
    return lax.axis_index("c") * _NS + lax.axis_index("s")


def _sc_mesh():
    return plsc.VectorSubcoreMesh(core_axis_name="c", subcore_axis_name="s",
                                  num_cores=_NC, num_subcores=_NS)


def _iota16():
    return lax.iota(jnp.int32, 16)


def _lgather(v, idx):
    return v.at[idx].get(mode='promise_in_bounds')


def _bmax8(v):
    for r in (4, 2, 1):
        idx = jnp.bitwise_and(_iota16() + r, 7)
        v = jnp.maximum(v, _lgather(v, idx))
    return v


def _bsum8(v):
    for r in (4, 2, 1):
        idx = jnp.bitwise_and(_iota16() + r, 7)
        v = v + _lgather(v, idx)
    return v


def _bcast(v, k):
    return _lgather(v, jnp.full((16,), k, jnp.int32))


def _pad16(v):
    return jnp.bitwise_and(v + 15, -16)


def _b1_body(src_hbm, dst_hbm, counts_hbm, sbuf, dbuf, cntbuf, cnt_s):
    w = _wid()

    def zero(b, _):
        cnt_s[b] = 0
        return 0
    lax.fori_loop(0, _NB, zero, 0)

    base = w * _EPW

    def chunk(j, _):
        pltpu.sync_copy(src_hbm.at[pl.ds(base + j * _KCH, _KCH)], sbuf)
        pltpu.sync_copy(dst_hbm.at[pl.ds(base + j * _KCH, _KCH)], dbuf)

        def grp(g, _):
            sv = sbuf[pl.ds(g * 16, 16)]
            dv = dbuf[pl.ds(g * 16, 16)]
            bv = lax.shift_right_logical(dv * _MAGIC, 22)
            tv = jnp.where(sv != dv, bv, _NB)
            for k in range(16):
                b = tv[k]
                cnt_s[b] = cnt_s[b] + 1
            return 0
        lax.fori_loop(0, _KCH // 16, grp, 0)
        return 0
    lax.fori_loop(0, _NCHP, chunk, 0)

    for i4 in range(_NB // 16):
        acc = jnp.zeros((16,), jnp.int32)
        for k in range(16):
            acc = jnp.where(_iota16() == k,
                            jnp.full((16,), cnt_s[i4 * 16 + k], jnp.int32),
                            acc)
        cntbuf[pl.ds(i4 * 16, 16)] = acc
    pltpu.sync_copy(cntbuf, counts_hbm.at[pl.ds(w * _NB, _NB)])


def _bin_count(src_pad, dst_pad):
    return pl.kernel(
        _b1_body,
        out_type=jax.ShapeDtypeStruct((_NW * _NB,), jnp.int32),
        mesh=_sc_mesh(),
        scratch_types=[
            pltpu.VMEM((_KCH,), jnp.int32),
            pltpu.VMEM((_KCH,), jnp.int32),
            pltpu.VMEM((_NB,), jnp.int32),
            pltpu.SMEM((_NB + 1,), jnp.int32),
        ],
    )(src_pad, dst_pad)


def _b2_body(src_hbm, dst_hbm, counts_hbm, bpk_hbm,
             sbuf, dbuf, cbuf, staging,
             gseg_s, lseg_s, fill_s, mini_s, semf):
    w = _wid()
    pltpu.sync_copy(counts_hbm, cbuf)

    for bg in range(_NB // 16):
        acc0 = (_iota16() + bg * 16) * _CAP

        def accp(p, a):
            return a + _pad16(cbuf[pl.ds(p * _NB + bg * 16, 16)])
        acc = lax.fori_loop(0, w, accp, acc0)
        myc = _pad16(cbuf[pl.ds(w * _NB + bg * 16, 16)])
        for k in range(16):
            gseg_s[bg * 16 + k] = acc[k]
            lseg_s[bg * 16 + k] = myc[k]

    def lpre(b, run):
        t = lseg_s[b]
        lseg_s[b] = run
        return run + t
    lax.fori_loop(0, _NB, lpre, 0)

    def zf(b, _):
        fill_s[b] = 0
        return 0
    lax.fori_loop(0, _NB + 1, zf, 0)

    base = w * _EPW

    def chunk(j, _):
        pltpu.sync_copy(src_hbm.at[pl.ds(base + j * _KCH, _KCH)], sbuf)
        pltpu.sync_copy(dst_hbm.at[pl.ds(base + j * _KCH, _KCH)], dbuf)

        def grp(g, _):
            sv = sbuf[pl.ds(g * 16, 16)]
            dv = dbuf[pl.ds(g * 16, 16)]
            bv = lax.shift_right_logical(dv * _MAGIC, 22)
            tv = jnp.where(sv != dv, bv, _NB)
            pkv = jnp.bitwise_or(lax.shift_left(sv, 14), dv)
            for k in range(16):
                b1 = tv[k]
                fc = fill_s[b1]
                mini_s[b1 * 16 + jnp.bitwise_and(fc, 15)] = pkv[k]
                fill_s[b1] = fc + 1
                do_flush = jnp.logical_and(jnp.bitwise_and(fc, 15) == 15,
                                           b1 < _NB)

                @pl.when(do_flush)
                def _():
                    acc = jnp.zeros((16,), jnp.int32)
                    for kk in range(16):
                        acc = jnp.where(
                            _iota16() == kk,
                            jnp.full((16,), mini_s[b1 * 16 + kk], jnp.int32),
                            acc)
                    spos = lseg_s[b1] + jnp.bitwise_and(fc, -16)
                    staging[pl.ds(spos, 16)] = acc
            return 0
        lax.fori_loop(0, _KCH // 16, grp, 0)
        return 0
    lax.fori_loop(0, _NCHP, chunk, 0)

    def tail(b, _):
        fc = fill_s[b]
        rem = jnp.bitwise_and(fc, 15)

        @pl.when(rem > 0)
        def _():
            sent = b * 157 * 16385
            acc = jnp.zeros((16,), jnp.int32)
            for kk in range(16):
                val = jnp.where(kk < rem, mini_s[b * 16 + kk], sent)
                acc = jnp.where(_iota16() == kk,
                                jnp.full((16,), val, jnp.int32), acc)
            spos = lseg_s[b] + jnp.bitwise_and(fc, -16)
            staging[pl.ds(spos, 16)] = acc
        return 0
    lax.fori_loop(0, _NB, tail, 0)

    def fire(b, _):
        nb16 = lax.shift_right_logical(fill_s[b] + 15, 4)
        ls = pl.multiple_of(lseg_s[b], 16)
        gs = pl.multiple_of(gseg_s[b], 16)

        def one(m, _):
            pltpu.async_copy(
                staging.at[pl.ds(ls + m * 16, 16)],
                bpk_hbm.at[pl.ds(gs + m * 16, 16)], semf)
            return 0
        lax.fori_loop(0, nb16, one, 0)
        return 0
    lax.fori_loop(0, _NB, fire, 0)

    def drain(b, _):
        nb16 = lax.shift_right_logical(fill_s[b] + 15, 4)
        ls = pl.multiple_of(lseg_s[b], 16)
        gs = pl.multiple_of(gseg_s[b], 16)

        def one(m, _):
            pltpu.make_async_copy(
                staging.at[pl.ds(ls + m * 16, 16)],
                bpk_hbm.at[pl.ds(gs + m * 16, 16)], semf).wait()
            return 0
        lax.fori_loop(0, nb16, one, 0)
        return 0
    lax.fori_loop(0, _NB, drain, 0)


def _bin_place(src_pad, dst_pad, counts):
    return pl.kernel(
        _b2_body,
        out_type=jax.ShapeDtypeStruct((_BE,), jnp.int32),
        mesh=_sc_mesh(),
        scratch_types=[
            pltpu.VMEM((_KCH,), jnp.int32),
            pltpu.VMEM((_KCH,), jnp.int32),
            pltpu.VMEM((_NW * _NB,), jnp.int32),
            pltpu.VMEM((_STG,), jnp.int32),
            pltpu.SMEM((_NB,), jnp.int32),
            pltpu.SMEM((_NB,), jnp.int32),
            pltpu.SMEM((_NB + 1,), jnp.int32),
            pltpu.SMEM(((_NB + 1) * 16,), jnp.int32),
            pltpu.SemaphoreType.DMA,
        ],
    )(src_pad, dst_pad, counts)


def _conv_body(bpk, counts_hbm, t_hbm, a_hbm, zrows0, drows0,
               z_hbm, deg_hbm, pbuf, idxs, dlbuf, webuf, tr,
               pbuf2, idxs2, dlbuf2, webuf2, tr2, adst,
               zloc, degloc, cbuf, tots_s, sem, sem2, *, cdim, with_deg):
    w = _wid()
    hc = _H * cdim
    pltpu.sync_copy(counts_hbm, cbuf)

    bg = (2 * w) // 16

    def accp(p, a):
        return a + _pad16(cbuf[pl.ds(p * _NB + bg * 16, 16)])
    tots16 = lax.fori_loop(0, _NW, accp, jnp.zeros((16,), jnp.int32))
    for k in range(16):
        tots_s[k] = tots16[k]

    onehot0 = jnp.where(_iota16() == 0, 1.0, 0.0).astype(jnp.float32)
    head_m = _iota16() < _H

    def bucket(bk, _):
        b = 2 * w + bk
        ptot = tots_s[b - bg * 16]

        pltpu.sync_copy(zrows0, zloc)
        if with_deg:
            pltpu.sync_copy(drows0, degloc)
        pltpu.sync_copy(a_hbm.at[pl.ds(b * _RNG * 16, _RNG * 16)], adst)

        nch = lax.shift_right_logical(ptot + _KCH - 1, 7)

        def fire(j, pbuf_r, idxs_r, dlbuf_r, webuf_r, tr_r, sem_r):
            st = b * _CAP + j * _KCH
            pltpu.sync_copy(bpk.at[pl.ds(st, _KCH)], pbuf_r)

            def sgrp(g, _):
                pos = _iota16() + (g * 16 + j * _KCH)
                mval = pos < ptot
                pk = pbuf_r[pl.ds(g * 16, 16)]
                sv = lax.shift_right_logical(pk, 14)
                dv = jnp.bitwise_and(pk, 16383)
                wev = jnp.logical_and(mval, sv != dv)
                webuf_r[pl.ds(g * 16, 16)] = jnp.where(wev, 1.0, 0.0)
                idxs_r[pl.ds(g * 16, 16)] = jnp.where(mval, sv, 0)
                dlbuf_r[pl.ds(g * 16, 16)] = jnp.clip(dv - b * _RNG, 0,
                                                      _RNG - 1)
                return 0
            lax.fori_loop(0, _KCH // 16, sgrp, 0)
            pltpu.async_copy(t_hbm.at[idxs_r], tr_r, sem_r)

        def wait(idxs_r, tr_r, sem_r):
            pltpu.make_async_copy(t_hbm.at[idxs_r], tr_r, sem_r).wait()

        def proc(tr, dlbuf, webuf):
            def egrp(g, _):
                dv16 = dlbuf[pl.ds(g * 16, 16)]
                wev16 = webuf[pl.ds(g * 16, 16)]
                for k in range(16):
                    e = g * 16 + k
                    dl = dv16[k]
                    lv = (tr[e, pl.ds(cdim, 16)]
                          - adst[pl.ds(dl * 16, 16)])
                    mx = _bmax8(jnp.where(head_m, lv, -1e30))
                    ev = jnp.where(head_m, jnp.exp(lv - mx), 0.0)
                    s = _bsum8(ev)
                    wk = _bcast(wev16, k)
                    qall = ev * (wk / s)
                    hvs = [tr[e, pl.ds(16 * k2, 16)]
                           for k2 in range(cdim // 16)]
                    zb = dl * hc
                    for hh in range(_H):
                        qv = _bcast(qall, hh)
                        for k2 in range(cdim // 16):
                            plsc.addupdate(
                                zloc.at[pl.ds(zb + hh * cdim + 16 * k2, 16)],
                                qv * hvs[k2])
                    if with_deg:
                        plsc.addupdate(degloc.at[pl.ds(dl * 16, 16)],
                                       onehot0 * wk)
                return 0
            lax.fori_loop(0, _KCH // 16, egrp, 0)

        fire(0, pbuf, idxs, dlbuf, webuf, tr, sem)

        def chunk(j, _):
            jp = jnp.minimum(j + 1, jnp.maximum(nch - 1, 0))

            @pl.when(jnp.bitwise_and(j, 1) == 0)
            def _():
                wait(idxs, tr, sem)
                fire(jp, pbuf2, idxs2, dlbuf2, webuf2, tr2, sem2)
                proc(tr, dlbuf, webuf)

            @pl.when(jnp.bitwise_and(j, 1) == 1)
            def _():
                wait(idxs2, tr2, sem2)
                fire(jp, pbuf, idxs, dlbuf, webuf, tr, sem)
                proc(tr2, dlbuf2, webuf2)
            return 0
        lax.fori_loop(0, nch, chunk, 0)

        @pl.when(jnp.bitwise_and(nch, 1) == 0)
        def _():
            wait(idxs, tr, sem)

        @pl.when(jnp.bitwise_and(nch, 1) == 1)
        def _():
            wait(idxs2, tr2, sem2)

        pltpu.sync_copy(zloc, z_hbm.at[pl.ds(b * _RNG * hc, _RNG * hc)])
        if with_deg:
            pltpu.sync_copy(degloc,
                            deg_hbm.at[pl.ds(b * _RNG * 16, _RNG * 16)])
        return 0
    lax.fori_loop(0, 2, bucket, 0)


def _edge_agg_sc(bpk, counts, t_tab, a_tab, cdim, with_deg):
    hc = _H * cdim
    zrows0 = jnp.zeros((_RNG * hc,), jnp.float32)
    drows0 = jnp.zeros((_RNG * 16,), jnp.float32)
    out_type = (jax.ShapeDtypeStruct((_NPAD * hc,), jnp.float32),
                jax.ShapeDtypeStruct((_NPAD * 16,), jnp.float32))
    scratch = [
        pltpu.VMEM((_KCH,), jnp.int32),
        pltpu.VMEM((_KCH,), jnp.int32),
        pltpu.VMEM((_KCH,), jnp.int32),
        pltpu.VMEM((_KCH,), jnp.float32),
        pltpu.VMEM((_KCH, 128), jnp.float32),
        pltpu.VMEM((_KCH,), jnp.int32),
        pltpu.VMEM((_KCH,), jnp.int32),
        pltpu.VMEM((_KCH,), jnp.int32),
        pltpu.VMEM((_KCH,), jnp.float32),
        pltpu.VMEM((_KCH, 128), jnp.float32),
        pltpu.VMEM((_RNG * 16,), jnp.float32),
        pltpu.VMEM((_RNG * hc,), jnp.float32),
        pltpu.VMEM((_RNG * 16,), jnp.float32),
        pltpu.VMEM((_NW * _NB,), jnp.int32),
        pltpu.SMEM((16,), jnp.int32),
        pltpu.SemaphoreType.DMA,
        pltpu.SemaphoreType.DMA,
    ]
    body = functools.partial(_conv_body, cdim=cdim, with_deg=with_deg)
    z, deg = pl.kernel(
        body, out_type=out_type, mesh=_sc_mesh(),
        scratch_types=scratch,
    )(bpk, counts, t_tab, a_tab, zrows0, drows0)
    return z.reshape(_NPAD, hc), deg.reshape(_NPAD, 16)


def _elu(v):
    return jnp.where(v > 0, v, jnp.exp(jnp.minimum(v, 0.0)) - 1.0)


def _ttab(h, a_plus_c, cdim):
    n = h.shape[0]
    pad = jnp.zeros((n, 128 - cdim - _H), jnp.float32)
    return jnp.concatenate([h, a_plus_c, pad], axis=1)


def _stage_in_body(x_ref, w_ref, b_ref, u_ref, c_ref, h_ref, t_ref, a_ref,
                   *, cdim):
    h = _elu(jnp.dot(x_ref[...], w_ref[...],
                     preferred_element_type=jnp.float32) + b_ref[...])
    h_ref[...] = h
    a = jnp.dot(h, u_ref[...], preferred_element_type=jnp.float32)
    t_ref[...] = _ttab(h, a + c_ref[...], cdim)
    a_ref[...] = jnp.concatenate(
        [a, jnp.zeros((h.shape[0], 16 - _H), jnp.float32)], axis=1)


def _stage_in(x, w, b, u, c):
    n = x.shape[0]
    cdim = w.shape[1]
    return pl.pallas_call(
        functools.partial(_stage_in_body, cdim=cdim),
        out_shape=(
            jax.ShapeDtypeStruct((n, cdim), jnp.float32),
            jax.ShapeDtypeStruct((n, 128), jnp.float32),
            jax.ShapeDtypeStruct((n, 16), jnp.float32),
        ),
    )(x, w, b.reshape(1, -1), u, c.reshape(1, -1))


def _stage_mid_body(z_ref, hprev_ref, deg_ref, wr_ref, b_ref, c_ref,
                    u_ref, cn_ref, h_ref, t_ref, a_ref, *, heads_c):
    heads, cin, cout = heads_c
    ql = jax.nn.softmax(c_ref[...][:, :heads], axis=-1)
    wr = wr_ref[...]
    weff = jnp.sum(wr.reshape(heads, cin, cout) * ql.reshape(heads, 1, 1),
                   axis=0)
    agg = (jnp.dot(z_ref[...], wr, preferred_element_type=jnp.float32)
           + jnp.dot(hprev_ref[...], weff, preferred_element_type=jnp.float32))
    cnt = 1.0 + deg_ref[...][:, 0:1]
    agg = agg / jnp.clip(cnt, 1.0, None)
    h = _elu(agg + b_ref[...])
    h_ref[...] = h
    a = jnp.dot(h, u_ref[...], preferred_element_type=jnp.float32)
    t_ref[...] = _ttab(h, a + cn_ref[...], cout)
    a_ref[...] = jnp.concatenate(
        [a, jnp.zeros((h.shape[0], 16 - _H), jnp.float32)], axis=1)


def _stage_mid(z, hprev, deg, wr, b, c, u_next, c_next):
    n = z.shape[0]
    heads_c = (_H, hprev.shape[1], wr.shape[1])
    return pl.pallas_call(
        functools.partial(_stage_mid_body, heads_c=heads_c),
        out_shape=(
            jax.ShapeDtypeStruct((n, wr.shape[1]), jnp.float32),
            jax.ShapeDtypeStruct((n, 128), jnp.float32),
            jax.ShapeDtypeStruct((n, 16), jnp.float32),
        ),
    )(z, hprev, deg, wr, b.reshape(1, -1), c.reshape(1, -1), u_next,
      c_next.reshape(1, -1))


def _stage_out_body(z_ref, hprev_ref, deg_ref, wr_ref, b_ref, c_ref,
                    w1_ref, b1_ref, w2_ref, b2_ref, o_ref, *, heads_c):
    heads, cin, cout = heads_c
    ql = jax.nn.softmax(c_ref[...][:, :heads], axis=-1)
    wr = wr_ref[...]
    weff = jnp.sum(wr.reshape(heads, cin, cout) * ql.reshape(heads, 1, 1),
                   axis=0)
    agg = (jnp.dot(z_ref[...], wr, preferred_element_type=jnp.float32)
           + jnp.dot(hprev_ref[...], weff, preferred_element_type=jnp.float32))
    cnt = 1.0 + deg_ref[...][:, 0:1]
    agg = agg / jnp.clip(cnt, 1.0, None)
    h = _elu(agg + b_ref[...])
    h = _elu(jnp.dot(h, w1_ref[...], preferred_element_type=jnp.float32)
             + b1_ref[...])
    logits = (jnp.dot(h, w2_ref[...], preferred_element_type=jnp.float32)
              + b2_ref[...])
    m = jnp.max(logits, axis=-1, keepdims=True)
    s = jnp.log(jnp.sum(jnp.exp(logits - m), axis=-1, keepdims=True))
    o_ref[...] = logits - m - s


def _stage_out(z, hprev, deg, wr, b, c, fc1_w, fc1_b, fc2_w, fc2_b):
    n = z.shape[0]
    heads_c = (_H, hprev.shape[1], wr.shape[1])
    return pl.pallas_call(
        functools.partial(_stage_out_body, heads_c=heads_c),
        out_shape=jax.ShapeDtypeStruct((n, fc2_w.shape[1]), jnp.float32),
    )(z, hprev, deg, wr, b.reshape(1, -1), c.reshape(1, -1), fc1_w,
      fc1_b.reshape(1, -1), fc2_w, fc2_b.reshape(1, -1))


def _rearrange_w(w, cin):
    out_c = w.shape[1] // _H
    return jnp.transpose(w.reshape(cin, _H, out_c), (1, 0, 2)).reshape(
        _H * cin, out_c)


def kernel(x, edge_index, fc0_w, fc0_b, w1, u1, c1, b1, w2, u2, c2, b2,
           w3, u3, c3, b3, fc1_w, fc1_b, fc2_w, fc2_b):
    src = jnp.pad(edge_index[0], (0, _EPAD - _E))
    dst = jnp.pad(edge_index[1], (0, _EPAD - _E))
    x_pad = jnp.pad(x, ((0, _NPAD - _N), (0, 0)))

    counts = _bin_count(src, dst)
    bpk = _bin_place(src, dst, counts)

    wr1 = _rearrange_w(w1, 16)
    wr2 = _rearrange_w(w2, 32)
    wr3 = _rearrange_w(w3, 64)

    h0, t1, a1 = _stage_in(x_pad, fc0_w, fc0_b, u1, c1)
    z1, deg = _edge_agg_sc(bpk, counts, t1, a1.reshape(-1), 16, True)
    h1, t2, a2 = _stage_mid(z1, h0, deg, wr1, b1, c1, u2, c2)
    z2, _ = _edge_agg_sc(bpk, counts, t2, a2.reshape(-1), 32, False)
    h2, t3, a3 = _stage_mid(z2, h1, deg, wr2, b2, c2, u3, c3)
    z3, _ = _edge_agg_sc(bpk, counts, t3, a3.reshape(-1), 64, False)
    out = _stage_out(z3, h2, deg, wr3, b3, c3, fc1_w, fc1_b, fc2_w, fc2_b)
    return out[:_N]
